# Initial kernel scaffold; baseline (speedup 1.0000x reference)
import jax
import jax.numpy as jnp
from jax.experimental import pallas as pl


def kernel(input, embed):
    raise NotImplementedError("write your pallas kernel here")


# speedup vs baseline: 1.1295x; 1.1295x over previous
import functools

import jax
import jax.numpy as jnp
from jax import lax
from jax.experimental import pallas as pl
from jax.experimental.pallas import tpu as pltpu
from jax.experimental.pallas import tpu_sc as plsc

_K = 8192
_D = 256
_BLK = 256
_COMMITMENT = 1.0


def _argmax_body(x_ref, e_ref, ind_ref, loss_ref, esq_ref):
    i = pl.program_id(0)
    x = x_ref[...]
    e = e_ref[...]

    @pl.when(i == 0)
    def _():
        esq_ref[...] = jnp.sum(e * e, axis=1)[None, :]
        loss_ref[...] = jnp.zeros((1, 1), jnp.float32)

    mm = lax.dot_general(x, e, (((1,), (1,)), ((), ())),
                         preferred_element_type=jnp.float32)
    xsq = jnp.sum(x * x, axis=1, keepdims=True)
    dist = -(xsq - 2.0 * mm + esq_ref[...])
    ind_ref[0, 0, :] = jnp.argmax(dist, axis=-1).astype(jnp.int32)
    best = jnp.max(dist, axis=-1, keepdims=True)
    loss_ref[...] -= jnp.sum(best, axis=0, keepdims=True)


def _dist_argmax(x, embed):
    n = x.shape[0]
    nblk = n // _BLK
    return pl.pallas_call(
        _argmax_body,
        grid=(nblk,),
        in_specs=[
            pl.BlockSpec((_BLK, _D), lambda i: (i, 0)),
            pl.BlockSpec((_K, _D), lambda i: (0, 0)),
        ],
        out_specs=[
            pl.BlockSpec((1, 1, _BLK), lambda i: (i, 0, 0)),
            pl.BlockSpec((1, 1), lambda i: (0, 0)),
        ],
        out_shape=[
            jax.ShapeDtypeStruct((nblk, 1, _BLK), jnp.int32),
            jax.ShapeDtypeStruct((1, 1), jnp.float32),
        ],
        scratch_shapes=[pltpu.VMEM((1, _K), jnp.float32)],
    )(x, embed)


def _gather_hist(embed, ind, n):
    info = plsc.get_sparse_core_info()
    nc, ns = info.num_cores, info.num_subcores
    nw = nc * ns
    bpw = n // nw

    mesh = plsc.VectorSubcoreMesh(core_axis_name="c", subcore_axis_name="s")

    @functools.partial(
        pl.kernel, mesh=mesh,
        out_type=[
            jax.ShapeDtypeStruct((n, _D), jnp.float32),
            jax.ShapeDtypeStruct((nc, _K), jnp.float32),
        ],
        scratch_types=[
            pltpu.VMEM((bpw,), jnp.int32),
            pltpu.VMEM((bpw, _D), jnp.float32),
            pltpu.VMEM((bpw,), jnp.float32),
            pltpu.VMEM((_K,), jnp.float32),
            pltpu.VMEM_SHARED((_K,), jnp.float32),
            pltpu.SemaphoreType.DMA,
        ],
    )
    def sc_body(embed_hbm, ind_hbm, quant_hbm, hist_hbm,
                idx_v, rows_v, ones_v, zero_v, hist_sh, sem):
        cid = lax.axis_index("c")
        sid = lax.axis_index("s")
        wid = sid * nc + cid
        base = wid * bpw
        pltpu.sync_copy(ind_hbm.at[pl.ds(base, bpw)], idx_v)
        pltpu.async_copy(embed_hbm.at[idx_v], rows_v, sem).wait()
        pltpu.sync_copy(rows_v, quant_hbm.at[pl.ds(base, bpw)])

        def fill(j, _):
            ones_v[pl.ds(j * 16, 16)] = jnp.ones((16,), jnp.float32)
            return _
        lax.fori_loop(0, bpw // 16, fill, 0)

        @pl.when(sid == 0)
        def _():
            def zero(j, _):
                zero_v[pl.ds(j * 16, 16)] = jnp.zeros((16,), jnp.float32)
                return _
            lax.fori_loop(0, _K // 16, zero, 0)
            pltpu.sync_copy(zero_v, hist_sh)

        plsc.subcore_barrier()
        pltpu.sync_copy(ones_v, hist_sh.at[idx_v], add=True)
        plsc.subcore_barrier()

        @pl.when(sid == 0)
        def _():
            pltpu.sync_copy(hist_sh, hist_hbm.at[cid])

    return sc_body(embed, ind)


def _finisher(hist_parts, loss_sum, n):
    inv_n = 1.0 / float(n)
    scale = _COMMITMENT / float(n * _D)

    def body(h_ref, l_ref, num_ref, closs_ref, perp_ref):
        h = jnp.sum(h_ref[...], axis=0, keepdims=True)
        num_ref[...] = h
        p = h * inv_n
        ent = jnp.sum(p * jnp.log(p + 1e-10), axis=1, keepdims=True)
        perp_ref[...] = jnp.exp(-ent)
        closs_ref[...] = l_ref[...] * scale

    nw = hist_parts.shape[0]
    return pl.pallas_call(
        body,
        in_specs=[
            pl.BlockSpec((nw, _K), lambda: (0, 0)),
            pl.BlockSpec((1, 1), lambda: (0, 0)),
        ],
        out_specs=[
            pl.BlockSpec((1, _K), lambda: (0, 0)),
            pl.BlockSpec((1, 1), lambda: (0, 0)),
            pl.BlockSpec((1, 1), lambda: (0, 0)),
        ],
        out_shape=[
            jax.ShapeDtypeStruct((1, _K), jnp.float32),
            jax.ShapeDtypeStruct((1, 1), jnp.float32),
            jax.ShapeDtypeStruct((1, 1), jnp.float32),
        ],
    )(hist_parts, loss_sum)


def kernel(input, embed):
    shape = input.shape
    x = input.reshape(-1, shape[-1])
    n = x.shape[0]

    ind3, loss_sum = _dist_argmax(x, embed)
    ind = ind3.reshape(n)
    quant_flat, hist_parts = _gather_hist(embed, ind, n)
    num2, closs, perp = _finisher(hist_parts, loss_sum, n)

    return (quant_flat.reshape(shape),
            num2.reshape(_K),
            closs.reshape(()),
            perp.reshape(()))
